# baseline (speedup 1.0000x reference)
import functools

import jax
import jax.numpy as jnp
from jax import lax
from jax.experimental import pallas as pl
from jax.experimental.pallas import tpu as pltpu
from jax.experimental.pallas import tpu_sc as plsc


def _sc_gather(table, idx, H, NT, D, NC, NS):
    NW = NC * NS
    b_per_w = H // NW
    mesh = plsc.VectorSubcoreMesh(core_axis_name="c", subcore_axis_name="s")

    @functools.partial(
        pl.kernel,
        mesh=mesh,
        out_type=jax.ShapeDtypeStruct((H, NT, D), jnp.float32),
        scratch_types=[
            pltpu.VMEM((b_per_w,), jnp.int32),
            pltpu.VMEM((b_per_w, NT, D), jnp.float32),
            pltpu.SemaphoreType.DMA,
        ],
    )
    def k(table_hbm, idx_hbm, out_hbm, idx_v, rows_v, sem):
        wid = lax.axis_index("s") * NC + lax.axis_index("c")
        base = wid * b_per_w
        pltpu.sync_copy(idx_hbm.at[pl.ds(base, b_per_w)], idx_v)
        pltpu.async_copy(table_hbm.at[idx_v], rows_v, sem).wait()
        pltpu.sync_copy(rows_v, out_hbm.at[pl.ds(base, b_per_w)])

    return k(table, idx)


def _tc_assemble_chunk(gathered, prefix, suffix, prev, B, H, NT, D, PL_, SL,
                       BB, block_off):
    T = PL_ + NT + SL

    def body(g_ref, p_ref, s_ref, *o_refs):
        o_ref = o_refs[-1]
        o_ref[:, 0:PL_, :] = jnp.broadcast_to(p_ref[...], (BB, PL_, D))
        o_ref[:, PL_:PL_ + NT, :] = g_ref[...]
        o_ref[:, PL_ + NT:T, :] = jnp.broadcast_to(s_ref[...], (BB, SL, D))

    in_specs = [
        pl.BlockSpec((BB, NT, D), lambda i: (i, 0, 0)),
        pl.BlockSpec((1, PL_, D), lambda i: (0, 0, 0)),
        pl.BlockSpec((1, SL, D), lambda i: (0, 0, 0)),
    ]
    operands = [gathered, prefix, suffix]
    aliases = {}
    if prev is not None:
        in_specs.append(pl.BlockSpec(memory_space=pl.ANY))
        operands.append(prev)
        aliases = {3: 0}

    return pl.pallas_call(
        body,
        grid=(H // BB,),
        in_specs=in_specs,
        out_specs=pl.BlockSpec((BB, T, D), lambda i: (i + block_off, 0, 0)),
        out_shape=jax.ShapeDtypeStruct((B, T, D), jnp.float32),
        input_output_aliases=aliases,
        compiler_params=pltpu.CompilerParams(
            dimension_semantics=("parallel",),
        ),
    )(*operands)


def kernel(label, cls_ctx, token_prefix, token_suffix):
    B = label.shape[0]
    V, NT, D = cls_ctx.shape
    PL_, SL = token_prefix.shape[1], token_suffix.shape[1]
    BB = 64
    H = B // 2

    idx = label.astype(jnp.int32)
    info = plsc.get_sparse_core_info()
    NC, NS = info.num_cores, info.num_subcores

    g1 = _sc_gather(cls_ctx, idx[:H], H, NT, D, NC, NS)
    g2 = _sc_gather(cls_ctx, idx[H:], H, NT, D, NC, NS)

    out1 = _tc_assemble_chunk(g1, token_prefix, token_suffix, None,
                              B, H, NT, D, PL_, SL, BB, block_off=0)
    out = _tc_assemble_chunk(g2, token_prefix, token_suffix, out1,
                             B, H, NT, D, PL_, SL, BB, block_off=H // BB)
    return out


# speedup vs baseline: 2.3001x; 2.3001x over previous
import functools

import jax
import jax.numpy as jnp
from jax import lax
from jax.experimental import pallas as pl
from jax.experimental.pallas import tpu as pltpu
from jax.experimental.pallas import tpu_sc as plsc


def _sc_gather(table, idx, B, NT, D, NC, NS):
    NW = NC * NS
    b_per_w = B // NW
    mesh = plsc.VectorSubcoreMesh(core_axis_name="c", subcore_axis_name="s")

    @functools.partial(
        pl.kernel,
        mesh=mesh,
        out_type=jax.ShapeDtypeStruct((B, NT, D), jnp.float32),
        scratch_types=[
            pltpu.VMEM((b_per_w,), jnp.int32),
            pltpu.VMEM((b_per_w, NT, D), jnp.float32),
            pltpu.SemaphoreType.DMA,
        ],
    )
    def k(table_hbm, idx_hbm, out_hbm, idx_v, rows_v, sem):
        wid = lax.axis_index("s") * NC + lax.axis_index("c")
        base = wid * b_per_w
        pltpu.sync_copy(idx_hbm.at[pl.ds(base, b_per_w)], idx_v)
        pltpu.async_copy(table_hbm.at[idx_v], rows_v, sem).wait()
        pltpu.sync_copy(rows_v, out_hbm.at[pl.ds(base, b_per_w)])

    return k(table, idx)


def _tc_assemble_tmajor(gathered, prefix_t, suffix_t, B, NT, D, PL_, SL, BB):
    T = PL_ + NT + SL

    def body(g_ref, p_ref, s_ref, o_ref):
        o_ref[0:PL_] = jnp.broadcast_to(p_ref[...], (PL_, BB, D))
        for t in range(NT):
            o_ref[PL_ + t] = g_ref[:, t, :]
        o_ref[PL_ + NT:T] = jnp.broadcast_to(s_ref[...], (SL, BB, D))

    return pl.pallas_call(
        body,
        grid=(B // BB,),
        in_specs=[
            pl.BlockSpec((BB, NT, D), lambda i: (i, 0, 0)),
            pl.BlockSpec((PL_, 1, D), lambda i: (0, 0, 0)),
            pl.BlockSpec((SL, 1, D), lambda i: (0, 0, 0)),
        ],
        out_specs=pl.BlockSpec((T, BB, D), lambda i: (0, i, 0)),
        out_shape=jax.ShapeDtypeStruct((T, B, D), jnp.float32),
        compiler_params=pltpu.CompilerParams(
            dimension_semantics=("parallel",),
        ),
    )(gathered, prefix_t, suffix_t)


def kernel(label, cls_ctx, token_prefix, token_suffix):
    B = label.shape[0]
    V, NT, D = cls_ctx.shape
    PL_, SL = token_prefix.shape[1], token_suffix.shape[1]

    idx = label.astype(jnp.int32)
    info = plsc.get_sparse_core_info()
    gathered = _sc_gather(cls_ctx, idx, B, NT, D,
                          info.num_cores, info.num_subcores)

    out_t = _tc_assemble_tmajor(
        gathered,
        jnp.transpose(token_prefix, (1, 0, 2)),
        jnp.transpose(token_suffix, (1, 0, 2)),
        B, NT, D, PL_, SL, BB=32,
    )
    return jnp.transpose(out_t, (1, 0, 2))


# speedup vs baseline: 2.3554x; 1.0240x over previous
import functools

import jax
import jax.numpy as jnp
from jax import lax
from jax.experimental import pallas as pl
from jax.experimental.pallas import tpu as pltpu
from jax.experimental.pallas import tpu_sc as plsc


def _sc_gather(table, idx, B, NT, D, NC, NS):
    NW = NC * NS
    b_per_w = B // NW
    mesh = plsc.VectorSubcoreMesh(core_axis_name="c", subcore_axis_name="s")

    @functools.partial(
        pl.kernel,
        mesh=mesh,
        out_type=jax.ShapeDtypeStruct((B, NT, D), jnp.float32),
        scratch_types=[
            pltpu.VMEM((b_per_w,), jnp.int32),
            pltpu.VMEM((b_per_w, NT, D), jnp.float32),
            pltpu.SemaphoreType.DMA,
        ],
    )
    def k(table_hbm, idx_hbm, out_hbm, idx_v, rows_v, sem):
        wid = lax.axis_index("s") * NC + lax.axis_index("c")
        base = wid * b_per_w
        pltpu.sync_copy(idx_hbm.at[pl.ds(base, b_per_w)], idx_v)
        pltpu.async_copy(table_hbm.at[idx_v], rows_v, sem).wait()
        pltpu.sync_copy(rows_v, out_hbm.at[pl.ds(base, b_per_w)])

    return k(table, idx)


def _tc_assemble_tmajor(gathered, prefix_t, suffix_t, B, NT, D, PL_, SL, BB):
    T = PL_ + NT + SL

    def body(g_ref, p_ref, s_ref, o_ref):
        o_ref[0:PL_] = jnp.broadcast_to(p_ref[...], (PL_, BB, D))
        for t in range(NT):
            o_ref[PL_ + t] = g_ref[:, t, :]
        o_ref[PL_ + NT:T] = jnp.broadcast_to(s_ref[...], (SL, BB, D))

    return pl.pallas_call(
        body,
        grid=(B // BB,),
        in_specs=[
            pl.BlockSpec((BB, NT, D), lambda i: (i, 0, 0)),
            pl.BlockSpec((PL_, 1, D), lambda i: (0, 0, 0)),
            pl.BlockSpec((SL, 1, D), lambda i: (0, 0, 0)),
        ],
        out_specs=pl.BlockSpec((T, BB, D), lambda i: (0, i, 0)),
        out_shape=jax.ShapeDtypeStruct((T, B, D), jnp.float32),
        compiler_params=pltpu.CompilerParams(
            dimension_semantics=("parallel",),
        ),
    )(gathered, prefix_t, suffix_t)


def kernel(label, cls_ctx, token_prefix, token_suffix):
    B = label.shape[0]
    V, NT, D = cls_ctx.shape
    PL_, SL = token_prefix.shape[1], token_suffix.shape[1]

    idx = label.astype(jnp.int32)
    info = plsc.get_sparse_core_info()
    gathered = _sc_gather(cls_ctx, idx, B, NT, D,
                          info.num_cores, info.num_subcores)

    out_t = _tc_assemble_tmajor(
        gathered,
        jnp.transpose(token_prefix, (1, 0, 2)),
        jnp.transpose(token_suffix, (1, 0, 2)),
        B, NT, D, PL_, SL, BB=64,
    )
    return jnp.transpose(out_t, (1, 0, 2))
